# baseline (speedup 1.0000x reference)
import functools

import jax
import jax.numpy as jnp
from jax import lax
from jax.experimental import pallas as pl
from jax.experimental.pallas import tpu as pltpu
from jax.experimental.pallas import tpu_sc as plsc


def _tapmm(x, w, b, taps, l_out, act, n_m=1):
    B, P, Lp, Cin = x.shape
    T, _, Cout = w.shape
    Q = len(taps)
    bm = l_out // n_m

    def body(x_ref, w_ref, b_ref, o_ref):
        for q in range(Q):
            for mb in range(n_m):
                acc = None
                for (p, off, t) in taps[q]:
                    a = x_ref[0, p, mb * bm + off:mb * bm + off + bm, :]
                    c = jnp.dot(a, w_ref[t], preferred_element_type=jnp.float32)
                    acc = c if acc is None else acc + c
                acc = acc + b_ref[0:1, :]
                if act == "relu":
                    acc = jnp.maximum(acc, 0.0)
                elif act == "tanh":
                    acc = jnp.tanh(acc)
                o_ref[0, q, mb * bm:(mb + 1) * bm, :] = acc

    return pl.pallas_call(
        body,
        grid=(B,),
        in_specs=[
            pl.BlockSpec((1, P, Lp, Cin), lambda i: (i, 0, 0, 0)),
            pl.BlockSpec((T, Cin, Cout), lambda i: (0, 0, 0)),
            pl.BlockSpec((1, Cout), lambda i: (0, 0)),
        ],
        out_specs=pl.BlockSpec((1, Q, l_out, Cout), lambda i: (i, 0, 0, 0)),
        out_shape=jax.ShapeDtypeStruct((B, Q, l_out, Cout), jnp.float32),
    )(x, w, b)


def _vq(zf, cb_t, cb_sq):
    TB = 512
    NB = zf.shape[0] // TB
    NC = cb_t.shape[1]

    def body(z_ref, c_ref, s_ref, i_ref):
        sc = jnp.dot(z_ref[...], c_ref[...], preferred_element_type=jnp.float32)
        d = s_ref[...] - 2.0 * sc
        m = jnp.min(d, axis=1, keepdims=True)
        ii = lax.broadcasted_iota(jnp.int32, d.shape, 1)
        idx = jnp.min(jnp.where(d == m, ii, jnp.int32(1 << 30)), axis=1)
        i_ref[...] = idx

    return pl.pallas_call(
        body,
        grid=(NB,),
        in_specs=[
            pl.BlockSpec((TB, zf.shape[1]), lambda i: (i, 0)),
            pl.BlockSpec(cb_t.shape, lambda i: (0, 0)),
            pl.BlockSpec((1, NC), lambda i: (0, 0)),
        ],
        out_specs=pl.BlockSpec((TB,), lambda i: (i,)),
        out_shape=jax.ShapeDtypeStruct((zf.shape[0],), jnp.int32),
    )(zf, cb_t, cb_sq)


def _gather_sc(idx, codebook):
    n_tok = idx.shape[0]
    d = codebook.shape[1]
    nw = 32
    per_w = n_tok // nw
    ch = 112
    n_ch = per_w // ch
    mesh = plsc.VectorSubcoreMesh(core_axis_name="c", subcore_axis_name="s")

    @functools.partial(
        pl.kernel,
        mesh=mesh,
        out_type=jax.ShapeDtypeStruct((n_tok, d), jnp.float32),
        scratch_types=[
            pltpu.VMEM((ch,), jnp.int32),
            pltpu.VMEM((ch, d), jnp.float32),
            pltpu.SemaphoreType.DMA,
        ],
    )
    def k(idx_hbm, table_hbm, out_hbm, idx_v, rows_v, sem):
        wid = lax.axis_index("s") * 2 + lax.axis_index("c")
        base = wid * per_w

        def chunk(c, carry):
            off = base + c * ch
            pltpu.sync_copy(idx_hbm.at[pl.ds(off, ch)], idx_v)
            pltpu.async_copy(table_hbm.at[idx_v], rows_v, sem).wait()
            pltpu.sync_copy(rows_v, out_hbm.at[pl.ds(off, ch)])
            return carry

        lax.fori_loop(0, n_ch, chunk, 0)

    return k(idx, codebook)


def _flat_pad(x_nhwc, pad_rows):
    B, H, W, C = x_nhwc.shape
    f = x_nhwc.reshape(B, H * W, C)
    return jnp.pad(f, ((0, 0), (0, pad_rows), (0, 0)))


def kernel(x, w1, b1, w2, b2, w3, b3, codebook, dw1, db1, dw2, db2, dw3, db3):
    f32 = jnp.float32

    xp = jnp.pad(jnp.transpose(x, (0, 2, 3, 1)), ((0, 0), (1, 1), (1, 1), (0, 0)))
    pats = [xp[:, kh:kh + 223:2, kw:kw + 223:2, :]
            for kh in range(4) for kw in range(4)]
    a1 = jnp.stack(pats, axis=3).reshape(8, 1, 12544, 48)
    wb1 = w1.transpose(2, 3, 1, 0).reshape(48, 128)[None]
    h1 = _tapmm(a1, wb1, b1[None, :], (((0, 0, 0),),), 12544, "relu")
    h1 = h1.reshape(8, 112, 112, 128)

    h1p = jnp.pad(h1, ((0, 0), (1, 1), (1, 1), (0, 0)))
    planes = jnp.stack(
        [h1p[:, pr::2, pc::2, :].reshape(8, 57 * 57, 128)
         for pr in range(2) for pc in range(2)], axis=1)
    planes = jnp.pad(planes, ((0, 0), (0, 0), (0, 7), (0, 0)))
    wb2 = w2.transpose(2, 3, 1, 0).reshape(16, 128, 256)
    taps2 = (tuple(((kh % 2) * 2 + (kw % 2), (kh // 2) * 57 + (kw // 2),
                    kh * 4 + kw) for kh in range(4) for kw in range(4)),)
    h2 = _tapmm(planes, wb2, b2[None, :], taps2, 56 * 57, "relu")
    h2 = h2.reshape(8, 56, 57, 256)[:, :, :56, :]

    h2p = jnp.pad(h2, ((0, 0), (1, 1), (1, 1), (0, 0)))
    h2f = _flat_pad(h2p, 4)[:, None]
    wb3 = w3.transpose(2, 3, 1, 0).reshape(9, 256, 256)
    taps3 = (tuple((0, kh * 58 + kw, kh * 3 + kw)
                   for kh in range(3) for kw in range(3)),)
    z = _tapmm(h2f, wb3, b3[None, :], taps3, 56 * 58, "none")
    zf = z.reshape(8, 56, 58, 256)[:, :, :56, :].reshape(25088, 256)

    ncb = codebook.shape[0]
    cb_t = jnp.pad(codebook.T, ((0, 0), (0, 1024 - ncb)))
    cb_sq = jnp.pad(jnp.sum(codebook * codebook, axis=1),
                    (0, 1024 - ncb), constant_values=1e30)[None, :]
    idx = _vq(zf, cb_t, cb_sq)
    e = _gather_sc(idx, codebook)
    zq = zf + lax.stop_gradient(e - zf)

    h3 = zq.reshape(8, 56, 56, 256)
    h3p = jnp.pad(h3, ((0, 0), (1, 1), (1, 1), (0, 0)))
    h3f = _flat_pad(h3p, 4)[:, None]
    wd1 = dw1.transpose(2, 3, 1, 0).reshape(9, 256, 256)
    h4 = _tapmm(h3f, wd1, db1[None, :], taps3, 56 * 58, "relu")
    h4 = h4.reshape(8, 56, 58, 256)[:, :, :56, :]

    h4p = jnp.pad(h4, ((0, 0), (1, 1), (1, 1), (0, 0)))
    h4f = _flat_pad(h4p, 4)[:, None]
    wd2 = dw2.transpose(2, 3, 1, 0).reshape(16, 256, 128)
    pairs = (((0, 0), (1, 2)), ((1, 1), (2, 3)))
    tapsT = tuple(
        tuple((0, dr * 58 + dc, kh * 4 + kw)
              for (dr, kh) in pairs[a] for (dc, kw) in pairs[b])
        for a in range(2) for b in range(2))
    h5 = _tapmm(h4f, wd2, db2[None, :], tapsT, 56 * 58, "relu")
    h5 = h5.reshape(8, 2, 2, 56, 58, 128)[:, :, :, :, :56, :]
    h5 = h5.transpose(0, 3, 1, 4, 2, 5).reshape(8, 112, 112, 128)

    h5p = jnp.pad(h5, ((0, 0), (1, 1), (1, 1), (0, 0)))
    h5f = _flat_pad(h5p, 8)[:, None]
    w16 = jnp.pad(dw3.transpose(2, 3, 1, 0),
                  ((0, 0), (0, 0), (0, 0), (0, 29)))
    zero = jnp.zeros((128, 32), f32)
    blocks = []
    tapsT3 = []
    for kh in range(4):
        for kw in range(4):
            q = (kh % 2) * 2 + (kw % 2)
            blocks.append(jnp.concatenate(
                [w16[kh, kw] if qq == q else zero for qq in range(4)], axis=1))
            tapsT3.append((0, ((kh + 1) // 2) * 114 + ((kw + 1) // 2),
                           kh * 4 + kw))
    wd3 = jnp.stack(blocks)
    db3p = jnp.tile(jnp.pad(db3, (0, 29)), 4)[None, :]
    y = _tapmm(h5f, wd3, db3p, (tuple(tapsT3),), 112 * 114, "tanh", n_m=4)
    y = y.reshape(8, 112, 114, 128)[:, :, :112, :]
    y = y.reshape(8, 112, 112, 2, 2, 32)
    y = y.transpose(0, 5, 1, 3, 2, 4).reshape(8, 32, 224, 224)
    x_hat = y[:, :3, :, :]

    return (x_hat, zf, e)


# speedup vs baseline: 1.4233x; 1.4233x over previous
import functools

import jax
import jax.numpy as jnp
from jax import lax
from jax.experimental import pallas as pl
from jax.experimental.pallas import tpu as pltpu
from jax.experimental.pallas import tpu_sc as plsc


def _tapmm(x, w, b, taps, l_out, act, n_m=1, out_dtype=jnp.float32):
    B, P, Lp, Cin = x.shape
    T, _, Cout = w.shape
    Q = len(taps)
    bm = l_out // n_m

    def body(x_ref, w_ref, b_ref, o_ref):
        for q in range(Q):
            for mb in range(n_m):
                acc = None
                for (p, off, t) in taps[q]:
                    a = x_ref[0, p, mb * bm + off:mb * bm + off + bm, :]
                    c = jnp.dot(a, w_ref[t], preferred_element_type=jnp.float32)
                    acc = c if acc is None else acc + c
                acc = acc + b_ref[0:1, :]
                if act == "relu":
                    acc = jnp.maximum(acc, 0.0)
                elif act == "tanh":
                    acc = jnp.tanh(acc)
                o_ref[0, q, mb * bm:(mb + 1) * bm, :] = acc.astype(out_dtype)

    return pl.pallas_call(
        body,
        grid=(B,),
        in_specs=[
            pl.BlockSpec((1, P, Lp, Cin), lambda i: (i, 0, 0, 0)),
            pl.BlockSpec((T, Cin, Cout), lambda i: (0, 0, 0)),
            pl.BlockSpec((1, Cout), lambda i: (0, 0)),
        ],
        out_specs=pl.BlockSpec((1, Q, l_out, Cout), lambda i: (i, 0, 0, 0)),
        out_shape=jax.ShapeDtypeStruct((B, Q, l_out, Cout), out_dtype),
    )(x, w, b)


def _vq(zf, cb_t, cb_sq):
    TB = 512
    NB = zf.shape[0] // TB
    NC = cb_t.shape[1]

    def body(z_ref, c_ref, s_ref, i_ref):
        sc = jnp.dot(z_ref[...], c_ref[...], preferred_element_type=jnp.float32)
        d = s_ref[...] - 2.0 * sc
        m = jnp.min(d, axis=1, keepdims=True)
        ii = lax.broadcasted_iota(jnp.int32, d.shape, 1)
        idx = jnp.min(jnp.where(d == m, ii, jnp.int32(1 << 30)), axis=1)
        i_ref[...] = idx

    return pl.pallas_call(
        body,
        grid=(NB,),
        in_specs=[
            pl.BlockSpec((TB, zf.shape[1]), lambda i: (i, 0)),
            pl.BlockSpec(cb_t.shape, lambda i: (0, 0)),
            pl.BlockSpec((1, NC), lambda i: (0, 0)),
        ],
        out_specs=pl.BlockSpec((TB,), lambda i: (i,)),
        out_shape=jax.ShapeDtypeStruct((zf.shape[0],), jnp.int32),
    )(zf, cb_t, cb_sq)


def _gather_sc(idx, codebook):
    n_tok = idx.shape[0]
    d = codebook.shape[1]
    nw = 32
    per_w = n_tok // nw
    ch = 112
    n_ch = per_w // ch
    mesh = plsc.VectorSubcoreMesh(core_axis_name="c", subcore_axis_name="s")

    @functools.partial(
        pl.kernel,
        mesh=mesh,
        out_type=jax.ShapeDtypeStruct((n_tok, d), jnp.float32),
        scratch_types=[
            pltpu.VMEM((ch,), jnp.int32),
            pltpu.VMEM((ch, d), jnp.float32),
            pltpu.SemaphoreType.DMA,
        ],
    )
    def k(idx_hbm, table_hbm, out_hbm, idx_v, rows_v, sem):
        wid = lax.axis_index("s") * 2 + lax.axis_index("c")
        base = wid * per_w

        def chunk(c, carry):
            off = base + c * ch
            pltpu.sync_copy(idx_hbm.at[pl.ds(off, ch)], idx_v)
            pltpu.async_copy(table_hbm.at[idx_v], rows_v, sem).wait()
            pltpu.sync_copy(rows_v, out_hbm.at[pl.ds(off, ch)])
            return carry

        lax.fori_loop(0, n_ch, chunk, 0)

    return k(idx, codebook)


def _flat_pad(x_nhwc, pad_rows):
    B, H, W, C = x_nhwc.shape
    f = x_nhwc.reshape(B, H * W, C)
    return jnp.pad(f, ((0, 0), (0, pad_rows), (0, 0)))


def kernel(x, w1, b1, w2, b2, w3, b3, codebook, dw1, db1, dw2, db2, dw3, db3):
    f32 = jnp.float32
    bf16 = jnp.bfloat16

    xp = jnp.pad(jnp.transpose(x.astype(bf16), (0, 2, 3, 1)),
                 ((0, 0), (1, 1), (1, 1), (0, 0)))
    pats = [xp[:, kh:kh + 223:2, kw:kw + 223:2, :]
            for kh in range(4) for kw in range(4)]
    a1 = jnp.stack(pats, axis=3).reshape(8, 1, 12544, 48)
    wb1 = w1.transpose(2, 3, 1, 0).reshape(48, 128)[None].astype(bf16)
    h1 = _tapmm(a1, wb1, b1[None, :], (((0, 0, 0),),), 12544, "relu",
                out_dtype=bf16)
    h1 = h1.reshape(8, 112, 112, 128)

    h1p = jnp.pad(h1, ((0, 0), (1, 1), (1, 1), (0, 0)))
    planes = jnp.stack(
        [h1p[:, pr::2, pc::2, :].reshape(8, 57 * 57, 128)
         for pr in range(2) for pc in range(2)], axis=1)
    planes = jnp.pad(planes, ((0, 0), (0, 0), (0, 7), (0, 0)))
    wb2 = w2.transpose(2, 3, 1, 0).reshape(16, 128, 256).astype(bf16)
    taps2 = (tuple(((kh % 2) * 2 + (kw % 2), (kh // 2) * 57 + (kw // 2),
                    kh * 4 + kw) for kh in range(4) for kw in range(4)),)
    h2 = _tapmm(planes, wb2, b2[None, :], taps2, 56 * 57, "relu",
                out_dtype=bf16)
    h2 = h2.reshape(8, 56, 57, 256)[:, :, :56, :]

    h2p = jnp.pad(h2, ((0, 0), (1, 1), (1, 1), (0, 0)))
    h2f = _flat_pad(h2p, 4)[:, None]
    wb3 = w3.transpose(2, 3, 1, 0).reshape(9, 256, 256).astype(bf16)
    taps3 = (tuple((0, kh * 58 + kw, kh * 3 + kw)
                   for kh in range(3) for kw in range(3)),)
    z = _tapmm(h2f, wb3, b3[None, :], taps3, 56 * 58, "none")
    zf = z.reshape(8, 56, 58, 256)[:, :, :56, :].reshape(25088, 256)

    ncb = codebook.shape[0]
    cb_t = jnp.pad(codebook.T, ((0, 0), (0, 1024 - ncb)))
    cb_sq = jnp.pad(jnp.sum(codebook * codebook, axis=1),
                    (0, 1024 - ncb), constant_values=1e30)[None, :]
    idx = _vq(zf, cb_t, cb_sq)
    e = _gather_sc(idx, codebook)
    zq = zf + lax.stop_gradient(e - zf)

    h3 = zq.astype(bf16).reshape(8, 56, 56, 256)
    h3p = jnp.pad(h3, ((0, 0), (1, 1), (1, 1), (0, 0)))
    h3f = _flat_pad(h3p, 4)[:, None]
    wd1 = dw1.transpose(2, 3, 1, 0).reshape(9, 256, 256).astype(bf16)
    h4 = _tapmm(h3f, wd1, db1[None, :], taps3, 56 * 58, "relu",
                out_dtype=bf16)
    h4 = h4.reshape(8, 56, 58, 256)[:, :, :56, :]

    h4p = jnp.pad(h4, ((0, 0), (1, 1), (1, 1), (0, 0)))
    h4f = _flat_pad(h4p, 4)[:, None]
    wd2 = dw2.transpose(2, 3, 1, 0).reshape(16, 256, 128).astype(bf16)
    pairs = (((0, 0), (1, 2)), ((1, 1), (2, 3)))
    tapsT = tuple(
        tuple((0, dr * 58 + dc, kh * 4 + kw)
              for (dr, kh) in pairs[a] for (dc, kw) in pairs[b])
        for a in range(2) for b in range(2))
    h5 = _tapmm(h4f, wd2, db2[None, :], tapsT, 56 * 58, "relu",
                out_dtype=bf16)
    h5 = h5.reshape(8, 2, 2, 56, 58, 128)[:, :, :, :, :56, :]
    h5 = h5.transpose(0, 3, 1, 4, 2, 5).reshape(8, 112, 112, 128)

    h5p = jnp.pad(h5, ((0, 0), (1, 1), (1, 1), (0, 0)))
    h5f = _flat_pad(h5p, 8)[:, None]
    w16 = jnp.pad(dw3.transpose(2, 3, 1, 0),
                  ((0, 0), (0, 0), (0, 0), (0, 29))).astype(bf16)
    zero = jnp.zeros((128, 32), bf16)
    blocks = []
    tapsT3 = []
    for kh in range(4):
        for kw in range(4):
            q = (kh % 2) * 2 + (kw % 2)
            blocks.append(jnp.concatenate(
                [w16[kh, kw] if qq == q else zero for qq in range(4)], axis=1))
            tapsT3.append((0, ((kh + 1) // 2) * 114 + ((kw + 1) // 2),
                           kh * 4 + kw))
    wd3 = jnp.stack(blocks)
    db3p = jnp.tile(jnp.pad(db3, (0, 29)), 4)[None, :]
    y = _tapmm(h5f, wd3, db3p, (tuple(tapsT3),), 112 * 114, "tanh", n_m=4)
    y = y.reshape(8, 112, 114, 128)[:, :, :112, :]
    y = y.reshape(8, 112, 112, 2, 2, 32)
    y = y.transpose(0, 5, 1, 3, 2, 4).reshape(8, 32, 224, 224)
    x_hat = y[:, :3, :, :]

    return (x_hat, zf, e)


# speedup vs baseline: 1.7719x; 1.2449x over previous
import functools

import jax
import jax.numpy as jnp
from jax import lax
from jax.experimental import pallas as pl
from jax.experimental.pallas import tpu as pltpu
from jax.experimental.pallas import tpu_sc as plsc

BF16 = jnp.bfloat16
F32 = jnp.float32


def _mtapmm(x, masks, w, b, taps, l_out, act, pb, n_m=1, out_dtype=F32):
    B, P, L, Cin = x.shape
    T, _, Cout = w.shape
    Q = len(taps)
    bm = l_out // n_m
    padl = pb + L + pb

    def body(x_ref, m_ref, w_ref, b_ref, o_ref, s_ref):
        @pl.when(pl.program_id(0) == 0)
        def _():
            zpad = jnp.zeros((pb, Cin), x.dtype)
            for src in range(3):
                for p in range(P):
                    s_ref[src, p, 0:pb, :] = zpad
                    s_ref[src, p, pb + L:padl, :] = zpad

        for p in range(P):
            vals = x_ref[0, p]
            s_ref[0, p, pb:pb + L, :] = vals
            s_ref[1, p, pb:pb + L, :] = vals * m_ref[0]
            s_ref[2, p, pb:pb + L, :] = vals * m_ref[1]

        for q in range(Q):
            for mb in range(n_m):
                acc = None
                for (p, src, off, t) in taps[q]:
                    st = pb + off + mb * bm
                    a = s_ref[src, p, st:st + bm, :]
                    c = jnp.dot(a, w_ref[t], preferred_element_type=F32)
                    acc = c if acc is None else acc + c
                acc = acc + b_ref[0:1, :]
                if act == "relu":
                    acc = jnp.maximum(acc, 0.0)
                elif act == "tanh":
                    acc = jnp.tanh(acc)
                o_ref[0, q, mb * bm:(mb + 1) * bm, :] = acc.astype(out_dtype)

    return pl.pallas_call(
        body,
        grid=(B,),
        in_specs=[
            pl.BlockSpec((1, P, L, Cin), lambda i: (i, 0, 0, 0)),
            pl.BlockSpec((2, L, Cin), lambda i: (0, 0, 0)),
            pl.BlockSpec((T, Cin, Cout), lambda i: (0, 0, 0)),
            pl.BlockSpec((1, Cout), lambda i: (0, 0)),
        ],
        out_specs=pl.BlockSpec((1, Q, l_out, Cout), lambda i: (i, 0, 0, 0)),
        out_shape=jax.ShapeDtypeStruct((B, Q, l_out, Cout), out_dtype),
        scratch_shapes=[pltpu.VMEM((3, P, padl, Cin), x.dtype)],
    )(x, masks, w, b)


def _border_masks(l, w_img, cin):
    col = jnp.arange(l) % w_img
    left = jnp.where(col == 0, 0.0, 1.0).astype(BF16)
    right = jnp.where(col == w_img - 1, 0.0, 1.0).astype(BF16)
    return jnp.stack([jnp.broadcast_to(left[:, None], (l, cin)),
                      jnp.broadcast_to(right[:, None], (l, cin))])


def _tapmm(x, w, b, l_out, act, out_dtype=F32):
    B, P, Lp, Cin = x.shape
    T, _, Cout = w.shape

    def body(x_ref, w_ref, b_ref, o_ref):
        acc = jnp.dot(x_ref[0, 0], w_ref[0], preferred_element_type=F32)
        acc = acc + b_ref[0:1, :]
        if act == "relu":
            acc = jnp.maximum(acc, 0.0)
        o_ref[0, 0] = acc.astype(out_dtype)

    return pl.pallas_call(
        body,
        grid=(B,),
        in_specs=[
            pl.BlockSpec((1, P, Lp, Cin), lambda i: (i, 0, 0, 0)),
            pl.BlockSpec((T, Cin, Cout), lambda i: (0, 0, 0)),
            pl.BlockSpec((1, Cout), lambda i: (0, 0)),
        ],
        out_specs=pl.BlockSpec((1, 1, l_out, Cout), lambda i: (i, 0, 0, 0)),
        out_shape=jax.ShapeDtypeStruct((B, 1, l_out, Cout), out_dtype),
    )(x, w, b)


def _vq(zf, cb_t, cb_sq):
    TB = 512
    NB = zf.shape[0] // TB
    NC = cb_t.shape[1]

    def body(z_ref, c_ref, s_ref, i_ref):
        sc = jnp.dot(z_ref[...], c_ref[...], preferred_element_type=F32)
        d = s_ref[...] - 2.0 * sc
        m = jnp.min(d, axis=1, keepdims=True)
        ii = lax.broadcasted_iota(jnp.int32, d.shape, 1)
        idx = jnp.min(jnp.where(d == m, ii, jnp.int32(1 << 30)), axis=1)
        i_ref[...] = idx

    return pl.pallas_call(
        body,
        grid=(NB,),
        in_specs=[
            pl.BlockSpec((TB, zf.shape[1]), lambda i: (i, 0)),
            pl.BlockSpec(cb_t.shape, lambda i: (0, 0)),
            pl.BlockSpec((1, NC), lambda i: (0, 0)),
        ],
        out_specs=pl.BlockSpec((TB,), lambda i: (i,)),
        out_shape=jax.ShapeDtypeStruct((zf.shape[0],), jnp.int32),
    )(zf, cb_t, cb_sq)


def _gather_sc(idx, codebook):
    n_tok = idx.shape[0]
    d = codebook.shape[1]
    nw = 32
    per_w = n_tok // nw
    ch = 112
    n_ch = per_w // ch
    mesh = plsc.VectorSubcoreMesh(core_axis_name="c", subcore_axis_name="s")

    @functools.partial(
        pl.kernel,
        mesh=mesh,
        out_type=jax.ShapeDtypeStruct((n_tok, d), F32),
        scratch_types=[
            pltpu.VMEM((ch,), jnp.int32),
            pltpu.VMEM((ch, d), F32),
            pltpu.SemaphoreType.DMA,
        ],
    )
    def k(idx_hbm, table_hbm, out_hbm, idx_v, rows_v, sem):
        wid = lax.axis_index("s") * 2 + lax.axis_index("c")
        base = wid * per_w

        def chunk(c, carry):
            off = base + c * ch
            pltpu.sync_copy(idx_hbm.at[pl.ds(off, ch)], idx_v)
            pltpu.async_copy(table_hbm.at[idx_v], rows_v, sem).wait()
            pltpu.sync_copy(rows_v, out_hbm.at[pl.ds(off, ch)])
            return carry

        lax.fori_loop(0, n_ch, chunk, 0)

    return k(idx, codebook)


_K4 = {0: (1, -1), 1: (0, 0), 2: (1, 0), 3: (0, 1)}
_SRC = {-1: 2, 0: 0, 1: 1}


def kernel(x, w1, b1, w2, b2, w3, b3, codebook, dw1, db1, dw2, db2, dw3, db3):
    xp = jnp.pad(jnp.transpose(x.astype(BF16), (0, 2, 3, 1)),
                 ((0, 0), (1, 1), (1, 1), (0, 0)))
    pats = [xp[:, kh:kh + 223:2, kw:kw + 223:2, :]
            for kh in range(4) for kw in range(4)]
    a1 = jnp.stack(pats, axis=3).reshape(8, 1, 12544, 48)
    wb1 = w1.transpose(2, 3, 1, 0).reshape(48, 128)[None].astype(BF16)
    h1 = _tapmm(a1, wb1, b1[None, :], 12544, "relu", out_dtype=BF16)

    m56 = _border_masks(3136, 56, 128)
    m56w = _border_masks(3136, 56, 256)
    m112 = _border_masks(12544, 112, 128)

    planes = (h1.reshape(8, 56, 2, 56, 2, 128)
              .transpose(0, 2, 4, 1, 3, 5).reshape(8, 4, 3136, 128))
    wb2 = w2.transpose(2, 3, 1, 0).reshape(16, 128, 256).astype(BF16)
    taps2 = []
    for kh in range(4):
        s, dm = _K4[kh]
        for kw in range(4):
            t, dn = _K4[kw]
            taps2.append((s * 2 + t, _SRC[dn], dm * 56 + dn, kh * 4 + kw))
    h2 = _mtapmm(planes, m56, wb2, b2[None, :], (tuple(taps2),), 3136,
                 "relu", pb=64, out_dtype=BF16)

    wb3 = w3.transpose(2, 3, 1, 0).reshape(9, 256, 256).astype(BF16)
    taps3 = (tuple((0, _SRC[dc], dr * 56 + dc, (dr + 1) * 3 + (dc + 1))
                   for dr in (-1, 0, 1) for dc in (-1, 0, 1)),)
    z = _mtapmm(h2, m56w, wb3, b3[None, :], taps3, 3136, "none", pb=64)
    zf = z.reshape(25088, 256)

    ncb = codebook.shape[0]
    cb_t = jnp.pad(codebook.T, ((0, 0), (0, 1024 - ncb)))
    cb_sq = jnp.pad(jnp.sum(codebook * codebook, axis=1),
                    (0, 1024 - ncb), constant_values=1e30)[None, :]
    idx = _vq(zf, cb_t, cb_sq)
    e = _gather_sc(idx, codebook)
    h3 = (zf + (e - zf)).astype(BF16).reshape(8, 1, 3136, 256)

    wd1 = dw1.transpose(2, 3, 1, 0).reshape(9, 256, 256).astype(BF16)
    h4 = _mtapmm(h3, m56w, wd1, db1[None, :], taps3, 3136, "relu",
                 pb=64, out_dtype=BF16)

    wd2 = dw2.transpose(2, 3, 1, 0).reshape(16, 256, 128).astype(BF16)
    pairs = (((0, 0), (1, 2)), ((1, 1), (2, 3)))
    tapsT = tuple(
        tuple((0, _SRC[dc - 1], (dr - 1) * 56 + (dc - 1), kh * 4 + kw)
              for (dr, kh) in pairs[a] for (dc, kw) in pairs[b])
        for a in range(2) for b in range(2))
    h5 = _mtapmm(h4, m56w, wd2, db2[None, :], tapsT, 3136, "relu",
                 pb=64, out_dtype=BF16)
    h5 = (h5.reshape(8, 2, 2, 56, 56, 128).transpose(0, 3, 1, 4, 2, 5)
          .reshape(8, 1, 12544, 128))

    w16 = jnp.pad(dw3.transpose(2, 3, 1, 0),
                  ((0, 0), (0, 0), (0, 0), (0, 29))).astype(BF16)
    zero = jnp.zeros((128, 32), BF16)
    blocks = []
    tapsT3 = []
    for kh in range(4):
        for kw in range(4):
            q = (kh % 2) * 2 + (kw % 2)
            blocks.append(jnp.concatenate(
                [w16[kh, kw] if qq == q else zero for qq in range(4)], axis=1))
            tapsT3.append((0, _SRC[(kw + 1) // 2 - 1],
                           ((kh + 1) // 2 - 1) * 112 + ((kw + 1) // 2 - 1),
                           kh * 4 + kw))
    wd3 = jnp.stack(blocks)
    db3p = jnp.tile(jnp.pad(db3, (0, 29)), 4)[None, :]
    y = _mtapmm(h5, m112, wd3, db3p, (tuple(tapsT3),), 12544, "tanh",
                pb=120, n_m=4, out_dtype=BF16)
    y = y.reshape(8, 112, 112, 2, 2, 32).transpose(0, 5, 1, 3, 2, 4)
    x_hat = y.reshape(8, 32, 224, 224)[:, :3, :, :].astype(F32)

    return (x_hat, zf, e)
